# baseline (speedup 1.0000x reference)
import functools

import jax
import jax.numpy as jnp
from jax import lax
from jax.experimental import pallas as pl
from jax.experimental.pallas import tpu as pltpu
from jax.experimental.pallas import tpu_sc as plsc

K = 20
EPS = 1e-5
NEG_INF = float("-inf")


def _knn_proj_kernel(xb_ref, xk_ref, wu_ref, wv_ref,
                     idx_ref, u_ref, v_ref, s_ref, *, n_total, bn):
    xb = xb_ref[0]
    xk = xk_ref[0]
    inner = lax.dot_general(xk, xb, (((1,), (1,)), ((), ())),
                            preferred_element_type=jnp.float32)
    sq_cols = jnp.sum(xb * xb, axis=1, keepdims=True).T
    s_ref[...] = 2.0 * inner - sq_cols

    ii = lax.broadcasted_iota(jnp.int32, (bn, n_total), 1)
    for k in range(K):
        s = s_ref[...]
        m = jnp.max(s, axis=1, keepdims=True)
        am = jnp.min(jnp.where(s == m, ii, n_total), axis=1, keepdims=True)
        idx_ref[0, :, pl.ds(k, 1)] = am
        s_ref[...] = jnp.where(ii == am, NEG_INF, s)

    u_ref[0] = jnp.dot(xk, wu_ref[...], preferred_element_type=jnp.float32)
    v_ref[0] = jnp.dot(xk, wv_ref[...], preferred_element_type=jnp.float32)


def _run_knn_proj(x, wu, wv, bn):
    B, N, C = x.shape
    co = wu.shape[1]
    grid = (B, N // bn)
    return pl.pallas_call(
        functools.partial(_knn_proj_kernel, n_total=N, bn=bn),
        grid=grid,
        in_specs=[
            pl.BlockSpec((1, N, C), lambda b, nb: (b, 0, 0)),
            pl.BlockSpec((1, bn, C), lambda b, nb: (b, nb, 0)),
            pl.BlockSpec((C, co), lambda b, nb: (0, 0)),
            pl.BlockSpec((C, co), lambda b, nb: (0, 0)),
        ],
        out_specs=[
            pl.BlockSpec((1, bn, K), lambda b, nb: (b, nb, 0)),
            pl.BlockSpec((1, bn, co), lambda b, nb: (b, nb, 0)),
            pl.BlockSpec((1, bn, co), lambda b, nb: (b, nb, 0)),
        ],
        out_shape=[
            jax.ShapeDtypeStruct((B, N, K), jnp.int32),
            jax.ShapeDtypeStruct((B, N, co), jnp.float32),
            jax.ShapeDtypeStruct((B, N, co), jnp.float32),
        ],
        scratch_shapes=[pltpu.VMEM((bn, N), jnp.float32)],
    )(x, x, wu, wv)


def _sc_gather_y1(vflat, gidx_flat, u2, chunk=80):
    total, co = gidx_flat.shape[0], vflat.shape[1]
    info = plsc.get_sparse_core_info()
    nw = info.num_cores * info.num_subcores
    per_w = total // nw
    n_chunks = per_w // chunk
    pts = chunk // K
    ng = co // 16
    mesh = plsc.VectorSubcoreMesh(core_axis_name="c", subcore_axis_name="s")

    @functools.partial(
        pl.kernel,
        out_type=[
            jax.ShapeDtypeStruct((total, co), jnp.float32),
            jax.ShapeDtypeStruct((nw, 2, co), jnp.float32),
        ],
        mesh=mesh,
        scratch_types=[
            pltpu.VMEM((chunk,), jnp.int32),
            pltpu.VMEM((chunk,), jnp.int32),
            pltpu.VMEM((chunk, co), jnp.float32),
            pltpu.VMEM((chunk, co), jnp.float32),
            pltpu.VMEM((pts, co), jnp.float32),
            pltpu.VMEM((2, co), jnp.float32),
            pltpu.SemaphoreType.DMA,
            pltpu.SemaphoreType.DMA,
        ],
    )
    def gather_k(vflat_hbm, gidx_hbm, u2_hbm, out_hbm, part_hbm,
                 idx_v0, idx_v1, rows_v0, rows_v1, u_v, acc_v, sem0, sem1):
        wid = lax.axis_index("s") * info.num_cores + lax.axis_index("c")
        w_base = wid * per_w
        w_pbase = wid * (per_w // K)

        def start_gather(j, idx_v, rows_v, sem):
            base = w_base + lax.rem(j, n_chunks) * chunk
            pltpu.sync_copy(gidx_hbm.at[pl.ds(base, chunk)], idx_v)
            pltpu.async_copy(vflat_hbm.at[idx_v], rows_v, sem)

        def drain(rows_v, sem):
            pltpu.make_async_copy(vflat_hbm.at[pl.ds(0, chunk)],
                                  rows_v, sem).wait()

        def compute(j, rows_v, accs):
            base = w_base + j * chunk
            pbase = w_pbase + j * pts
            pltpu.sync_copy(u2_hbm.at[pl.ds(pbase, pts)], u_v)
            for p in range(pts):
                us = [u_v[p, pl.ds(g * 16, 16)] for g in range(ng)]

                def row_body(r, a):
                    row = p * K + r
                    na = list(a)
                    for g in range(ng):
                        y = rows_v[row, pl.ds(g * 16, 16)] + us[g]
                        rows_v[row, pl.ds(g * 16, 16)] = y
                        na[g] = na[g] + y
                        na[ng + g] = na[ng + g] + y * y
                    return tuple(na)

                accs = lax.fori_loop(0, K, row_body, accs, unroll=False)
            pltpu.sync_copy(rows_v, out_hbm.at[pl.ds(base, chunk)])
            return accs

        def pair_body(c, accs):
            j0 = 2 * c
            start_gather(j0 + 1, idx_v1, rows_v1, sem1)
            drain(rows_v0, sem0)
            accs = compute(j0, rows_v0, accs)
            start_gather(j0 + 2, idx_v0, rows_v0, sem0)
            drain(rows_v1, sem1)
            return compute(j0 + 1, rows_v1, accs)

        zeros = tuple(jnp.zeros((16,), jnp.float32) for _ in range(2 * ng))
        start_gather(0, idx_v0, rows_v0, sem0)
        accs = lax.fori_loop(0, n_chunks // 2, pair_body, zeros, unroll=False)
        drain(rows_v0, sem0)
        for g in range(ng):
            acc_v[0, pl.ds(g * 16, 16)] = accs[g]
            acc_v[1, pl.ds(g * 16, 16)] = accs[ng + g]
        pltpu.sync_copy(acc_v, part_hbm.at[wid])

    return gather_k(vflat, gidx_flat, u2)


def _mlp2_kernel(y1_ref, sc1_ref, sh1_ref, w2_ref,
                 sum_ref, sq_ref, mx_ref, mn_ref, *, r):
    k, co = y1_ref.shape[1], y1_ref.shape[2]
    y1 = y1_ref[...]
    h1 = sc1_ref[0][None, None, :] * y1 + sh1_ref[0][None, None, :]
    h1 = jnp.where(h1 >= 0.0, h1, 0.2 * h1)
    y2 = jnp.dot(h1.reshape(r * k, co), w2_ref[...],
                 preferred_element_type=jnp.float32).reshape(r, k, co)
    sum_ref[...] = jnp.sum(y2, axis=(0, 1), keepdims=True)
    sq_ref[...] = jnp.sum(y2 * y2, axis=(0, 1), keepdims=True)
    mx_ref[...] = jnp.max(y2, axis=1)
    mn_ref[...] = jnp.min(y2, axis=1)


def _run_mlp2(y13, scale1, shift1, w2t, r):
    M, k, co = y13.shape
    g = M // r
    return pl.pallas_call(
        functools.partial(_mlp2_kernel, r=r),
        grid=(g,),
        in_specs=[
            pl.BlockSpec((r, k, co), lambda i: (i, 0, 0)),
            pl.BlockSpec((1, co), lambda i: (0, 0)),
            pl.BlockSpec((1, co), lambda i: (0, 0)),
            pl.BlockSpec((co, co), lambda i: (0, 0)),
        ],
        out_specs=[
            pl.BlockSpec((1, 1, co), lambda i: (i, 0, 0)),
            pl.BlockSpec((1, 1, co), lambda i: (i, 0, 0)),
            pl.BlockSpec((r, co), lambda i: (i, 0)),
            pl.BlockSpec((r, co), lambda i: (i, 0)),
        ],
        out_shape=[
            jax.ShapeDtypeStruct((g, 1, co), jnp.float32),
            jax.ShapeDtypeStruct((g, 1, co), jnp.float32),
            jax.ShapeDtypeStruct((M, co), jnp.float32),
            jax.ShapeDtypeStruct((M, co), jnp.float32),
        ],
    )(y13, scale1, shift1, w2t)


def _final_kernel(mx_ref, mn_ref, sc2_ref, sh2_ref, out_ref):
    a = sc2_ref[0][None, :]
    pick = jnp.where(a >= 0.0, mx_ref[...], mn_ref[...])
    y = a * pick + sh2_ref[0][None, :]
    out_ref[...] = jnp.where(y >= 0.0, y, 0.2 * y)


def _run_final(mx, mn, scale2, shift2, r):
    M, co = mx.shape
    g = M // r
    return pl.pallas_call(
        _final_kernel,
        grid=(g,),
        in_specs=[
            pl.BlockSpec((r, co), lambda i: (i, 0)),
            pl.BlockSpec((r, co), lambda i: (i, 0)),
            pl.BlockSpec((1, co), lambda i: (0, 0)),
            pl.BlockSpec((1, co), lambda i: (0, 0)),
        ],
        out_specs=pl.BlockSpec((r, co), lambda i: (i, 0)),
        out_shape=jax.ShapeDtypeStruct((M, co), jnp.float32),
    )(mx, mn, scale2, shift2)


def kernel(pc_ftr, W1, b1, g1, be1, W2, b2, g2, be2):
    B, N, C = pc_ftr.shape
    co = W1.shape[0]
    n_edges = B * N * K

    wu = (W1[:, :C] - W1[:, C:]).T
    wv = W1[:, C:].T
    w2t = W2.T

    y1s, parts = [], []
    for b in range(B):
        gidx_b, u_b, v_b = _run_knn_proj(pc_ftr[b:b + 1], wu, wv, bn=512)
        y1_b, part_b = _sc_gather_y1(v_b.reshape(N, co),
                                     gidx_b.reshape(N * K),
                                     u_b.reshape(N, co))
        y1s.append(y1_b.reshape(N, K, co))
        parts.append(part_b)

    s1 = sum(p.sum(0) for p in parts)
    mean1 = s1[0] / n_edges
    var1 = s1[1] / n_edges - mean1 * mean1
    scale1 = g1 / jnp.sqrt(var1 + EPS)
    shift1 = (be1 - mean1 * scale1)

    sum2 = jnp.zeros((co,), jnp.float32)
    sq2 = jnp.zeros((co,), jnp.float32)
    mxs, mns = [], []
    for b in range(B):
        p_sum2, p_sq2, mx_b, mn_b = _run_mlp2(
            y1s[b], scale1[None, :], shift1[None, :], w2t, r=256)
        sum2 = sum2 + p_sum2.sum((0, 1))
        sq2 = sq2 + p_sq2.sum((0, 1))
        mxs.append(mx_b)
        mns.append(mn_b)
    mean2 = sum2 / n_edges
    var2 = sq2 / n_edges - mean2 * mean2
    scale2 = g2 / jnp.sqrt(var2 + EPS)
    shift2 = (be2 - mean2 * scale2)

    mx = jnp.concatenate(mxs, axis=0)
    mn = jnp.concatenate(mns, axis=0)
    out = _run_final(mx, mn, scale2[None, :], shift2[None, :], r=512)
    return out.reshape(B, N, co)


# speedup vs baseline: 1.0246x; 1.0246x over previous
import functools

import jax
import jax.numpy as jnp
from jax import lax
from jax.experimental import pallas as pl
from jax.experimental.pallas import tpu as pltpu
from jax.experimental.pallas import tpu_sc as plsc

K = 20
EPS = 1e-5
NEG_INF = float("-inf")


def _knn_proj_kernel(xb_ref, xk_ref, wu_ref, wv_ref,
                     idx_ref, u_ref, v_ref, s_ref, *, n_total, bn):
    xb = xb_ref[0]
    xk = xk_ref[0]
    inner = lax.dot_general(xk, xb, (((1,), (1,)), ((), ())),
                            preferred_element_type=jnp.float32)
    sq_cols = jnp.sum(xb * xb, axis=1, keepdims=True).T
    s_ref[...] = 2.0 * inner - sq_cols

    ii = lax.broadcasted_iota(jnp.int32, (bn, n_total), 1)
    for k in range(K):
        s = s_ref[...]
        m = jnp.max(s, axis=1, keepdims=True)
        am = jnp.min(jnp.where(s == m, ii, n_total), axis=1, keepdims=True)
        idx_ref[0, :, pl.ds(k, 1)] = am
        s_ref[...] = jnp.where(ii == am, NEG_INF, s)

    u_ref[0] = jnp.dot(xk, wu_ref[...], preferred_element_type=jnp.float32)
    v_ref[0] = jnp.dot(xk, wv_ref[...], preferred_element_type=jnp.float32)


def _run_knn_proj(x, wu, wv, bn):
    B, N, C = x.shape
    co = wu.shape[1]
    grid = (B, N // bn)
    return pl.pallas_call(
        functools.partial(_knn_proj_kernel, n_total=N, bn=bn),
        grid=grid,
        in_specs=[
            pl.BlockSpec((1, N, C), lambda b, nb: (b, 0, 0)),
            pl.BlockSpec((1, bn, C), lambda b, nb: (b, nb, 0)),
            pl.BlockSpec((C, co), lambda b, nb: (0, 0)),
            pl.BlockSpec((C, co), lambda b, nb: (0, 0)),
        ],
        out_specs=[
            pl.BlockSpec((1, bn, K), lambda b, nb: (b, nb, 0)),
            pl.BlockSpec((1, bn, co), lambda b, nb: (b, nb, 0)),
            pl.BlockSpec((1, bn, co), lambda b, nb: (b, nb, 0)),
        ],
        out_shape=[
            jax.ShapeDtypeStruct((B, N, K), jnp.int32),
            jax.ShapeDtypeStruct((B, N, co), jnp.float32),
            jax.ShapeDtypeStruct((B, N, co), jnp.float32),
        ],
        scratch_shapes=[pltpu.VMEM((bn, N), jnp.float32)],
    )(x, x, wu, wv)


def _sc_gather_y1(vflat, gidx_flat, u2, chunk=80):
    total, co = gidx_flat.shape[0], vflat.shape[1]
    info = plsc.get_sparse_core_info()
    nw = info.num_cores * info.num_subcores
    per_w = total // nw
    n_chunks = per_w // chunk
    pts = chunk // K
    ng = co // 16
    mesh = plsc.VectorSubcoreMesh(core_axis_name="c", subcore_axis_name="s")

    @functools.partial(
        pl.kernel,
        out_type=[
            jax.ShapeDtypeStruct((total, co), jnp.float32),
            jax.ShapeDtypeStruct((nw, 2, co), jnp.float32),
        ],
        mesh=mesh,
        scratch_types=[
            pltpu.VMEM((chunk,), jnp.int32),
            pltpu.VMEM((chunk,), jnp.int32),
            pltpu.VMEM((chunk, co), jnp.float32),
            pltpu.VMEM((chunk, co), jnp.float32),
            pltpu.VMEM((pts, co), jnp.float32),
            pltpu.VMEM((2, co), jnp.float32),
            pltpu.SemaphoreType.DMA,
            pltpu.SemaphoreType.DMA,
        ],
    )
    def gather_k(vflat_hbm, gidx_hbm, u2_hbm, out_hbm, part_hbm,
                 idx_v0, idx_v1, rows_v0, rows_v1, u_v, acc_v, sem0, sem1):
        wid = lax.axis_index("s") * info.num_cores + lax.axis_index("c")
        w_base = wid * per_w
        w_pbase = wid * (per_w // K)

        def start_gather(j, idx_v, rows_v, sem):
            base = w_base + lax.rem(j, n_chunks) * chunk
            pltpu.sync_copy(gidx_hbm.at[pl.ds(base, chunk)], idx_v)
            pltpu.async_copy(vflat_hbm.at[idx_v], rows_v, sem)

        def drain(rows_v, sem):
            pltpu.make_async_copy(vflat_hbm.at[pl.ds(0, chunk)],
                                  rows_v, sem).wait()

        def compute(j, rows_v, accs):
            base = w_base + j * chunk
            pbase = w_pbase + j * pts
            pltpu.sync_copy(u2_hbm.at[pl.ds(pbase, pts)], u_v)
            for p in range(pts):
                us = [u_v[p, pl.ds(g * 16, 16)] for g in range(ng)]

                def row_body(r, a):
                    row = p * K + r
                    na = list(a)
                    for g in range(ng):
                        y = rows_v[row, pl.ds(g * 16, 16)] + us[g]
                        rows_v[row, pl.ds(g * 16, 16)] = y
                        na[g] = na[g] + y
                        na[ng + g] = na[ng + g] + y * y
                    return tuple(na)

                accs = lax.fori_loop(0, K, row_body, accs, unroll=False)
            pltpu.sync_copy(rows_v, out_hbm.at[pl.ds(base, chunk)])
            return accs

        def pair_body(c, accs):
            j0 = 2 * c
            start_gather(j0 + 1, idx_v1, rows_v1, sem1)
            drain(rows_v0, sem0)
            accs = compute(j0, rows_v0, accs)
            start_gather(j0 + 2, idx_v0, rows_v0, sem0)
            drain(rows_v1, sem1)
            return compute(j0 + 1, rows_v1, accs)

        zeros = tuple(jnp.zeros((16,), jnp.float32) for _ in range(2 * ng))
        start_gather(0, idx_v0, rows_v0, sem0)
        accs = lax.fori_loop(0, n_chunks // 2, pair_body, zeros, unroll=False)
        drain(rows_v0, sem0)
        for g in range(ng):
            acc_v[0, pl.ds(g * 16, 16)] = accs[g]
            acc_v[1, pl.ds(g * 16, 16)] = accs[ng + g]
        pltpu.sync_copy(acc_v, part_hbm.at[wid])

    return gather_k(vflat, gidx_flat, u2)


def _mlp2_kernel(y1_ref, sc1_ref, sh1_ref, w2_ref,
                 sum_ref, sq_ref, mx_ref, mn_ref, *, r):
    k, co = y1_ref.shape[1], y1_ref.shape[2]
    y1 = y1_ref[...]
    h1 = sc1_ref[0][None, None, :] * y1 + sh1_ref[0][None, None, :]
    h1 = jnp.where(h1 >= 0.0, h1, 0.2 * h1)
    y2 = jnp.dot(h1.reshape(r * k, co), w2_ref[...],
                 preferred_element_type=jnp.float32).reshape(r, k, co)
    sum_ref[...] = jnp.sum(y2, axis=(0, 1), keepdims=True)
    sq_ref[...] = jnp.sum(y2 * y2, axis=(0, 1), keepdims=True)
    mx_ref[...] = jnp.max(y2, axis=1)
    mn_ref[...] = jnp.min(y2, axis=1)


def _run_mlp2(y13, scale1, shift1, w2t, r):
    M, k, co = y13.shape
    g = M // r
    return pl.pallas_call(
        functools.partial(_mlp2_kernel, r=r),
        grid=(g,),
        in_specs=[
            pl.BlockSpec((r, k, co), lambda i: (i, 0, 0)),
            pl.BlockSpec((1, co), lambda i: (0, 0)),
            pl.BlockSpec((1, co), lambda i: (0, 0)),
            pl.BlockSpec((co, co), lambda i: (0, 0)),
        ],
        out_specs=[
            pl.BlockSpec((1, 1, co), lambda i: (i, 0, 0)),
            pl.BlockSpec((1, 1, co), lambda i: (i, 0, 0)),
            pl.BlockSpec((r, co), lambda i: (i, 0)),
            pl.BlockSpec((r, co), lambda i: (i, 0)),
        ],
        out_shape=[
            jax.ShapeDtypeStruct((g, 1, co), jnp.float32),
            jax.ShapeDtypeStruct((g, 1, co), jnp.float32),
            jax.ShapeDtypeStruct((M, co), jnp.float32),
            jax.ShapeDtypeStruct((M, co), jnp.float32),
        ],
    )(y13, scale1, shift1, w2t)


def _final_kernel(mx_ref, mn_ref, sc2_ref, sh2_ref, out_ref):
    a = sc2_ref[0][None, :]
    pick = jnp.where(a >= 0.0, mx_ref[...], mn_ref[...])
    y = a * pick + sh2_ref[0][None, :]
    out_ref[...] = jnp.where(y >= 0.0, y, 0.2 * y)


def _run_final(mx, mn, scale2, shift2, r):
    M, co = mx.shape
    g = M // r
    return pl.pallas_call(
        _final_kernel,
        grid=(g,),
        in_specs=[
            pl.BlockSpec((r, co), lambda i: (i, 0)),
            pl.BlockSpec((r, co), lambda i: (i, 0)),
            pl.BlockSpec((1, co), lambda i: (0, 0)),
            pl.BlockSpec((1, co), lambda i: (0, 0)),
        ],
        out_specs=pl.BlockSpec((r, co), lambda i: (i, 0)),
        out_shape=jax.ShapeDtypeStruct((M, co), jnp.float32),
    )(mx, mn, scale2, shift2)


def kernel(pc_ftr, W1, b1, g1, be1, W2, b2, g2, be2):
    B, N, C = pc_ftr.shape
    co = W1.shape[0]
    n_edges = B * N * K

    wu = (W1[:, :C] - W1[:, C:]).T
    wv = W1[:, C:].T
    w2t = W2.T

    y1s, parts = [], []
    for b in range(B):
        gidx_b, u_b, v_b = _run_knn_proj(pc_ftr[b:b + 1], wu, wv, bn=256)
        y1_b, part_b = _sc_gather_y1(v_b.reshape(N, co),
                                     gidx_b.reshape(N * K),
                                     u_b.reshape(N, co))
        y1s.append(y1_b.reshape(N, K, co))
        parts.append(part_b)

    s1 = sum(p.sum(0) for p in parts)
    mean1 = s1[0] / n_edges
    var1 = s1[1] / n_edges - mean1 * mean1
    scale1 = g1 / jnp.sqrt(var1 + EPS)
    shift1 = (be1 - mean1 * scale1)

    sum2 = jnp.zeros((co,), jnp.float32)
    sq2 = jnp.zeros((co,), jnp.float32)
    mxs, mns = [], []
    for b in range(B):
        p_sum2, p_sq2, mx_b, mn_b = _run_mlp2(
            y1s[b], scale1[None, :], shift1[None, :], w2t, r=256)
        sum2 = sum2 + p_sum2.sum((0, 1))
        sq2 = sq2 + p_sq2.sum((0, 1))
        mxs.append(mx_b)
        mns.append(mn_b)
    mean2 = sum2 / n_edges
    var2 = sq2 / n_edges - mean2 * mean2
    scale2 = g2 / jnp.sqrt(var2 + EPS)
    shift2 = (be2 - mean2 * scale2)

    mx = jnp.concatenate(mxs, axis=0)
    mn = jnp.concatenate(mns, axis=0)
    out = _run_final(mx, mn, scale2[None, :], shift2[None, :], r=512)
    return out.reshape(B, N, co)
